# Initial kernel scaffold; baseline (speedup 1.0000x reference)
import jax
import jax.numpy as jnp
from jax.experimental import pallas as pl


def kernel(x, w1, m1, b1, w2, m2, b2, wk1, bk1, wk2, bk2):
    raise NotImplementedError("write your pallas kernel here")


# speedup vs baseline: 1.2912x; 1.2912x over previous
import functools

import jax
import jax.numpy as jnp
from jax.experimental import pallas as pl
from jax.experimental.pallas import tpu as pltpu

CONV_GRID_SIZE, CONV_SPLINE_ORDER = 5, 3
KAN_GRID_SIZE, KAN_SPLINE_ORDER = 3, 3
_C0, _C1, _C2 = 2, 8, 16
_BT = 8
_NT = 8


def _ext_grid(num, k, lo=-1.0, hi=1.0):
    h = (hi - lo) / num
    return tuple(lo + (i - k) * h for i in range(num + 2 * k + 1))


_CONV_GRID = _ext_grid(CONV_GRID_SIZE, CONV_SPLINE_ORDER)
_KAN_GRID = _ext_grid(KAN_GRID_SIZE, KAN_SPLINE_ORDER)


def _bsplines(x, grid_pts, order):
    g = grid_pts
    step = [jnp.where(x >= gi, 1.0, 0.0) for gi in g]
    bases = [step[i] - step[i + 1] for i in range(len(g) - 1)]
    for k in range(1, order + 1):
        bases = [
            (x - g[i]) * (1.0 / (g[i + k] - g[i])) * bases[i]
            + (g[i + k + 1] - x) * (1.0 / (g[i + k + 1] - g[i + 1])) * bases[i + 1]
            for i in range(len(g) - 1 - k)
        ]
    return bases


def _dot(a, b):
    return jnp.dot(a, b, preferred_element_type=jnp.float32)


def _kkan_body(x_ref, w1_ref, m1_ref, b1_ref, w2_ref, m2_ref, b2_ref,
               wk1_ref, bk1_ref, wk2_ref, bk2_ref, o_ref):
    bf = jnp.float32
    x = x_ref[...]

    s = x * jax.nn.sigmoid(x)
    bk = _bsplines(x, _CONV_GRID, CONV_SPLINE_ORDER)

    flats = []
    for t in range(_NT):
        r0 = t * _BT * _C0
        sl = slice(r0, r0 + _BT * _C0)
        f1 = jnp.concatenate([s[sl]] + [b[sl] for b in bk], axis=0)
        y1 = _dot(w1_ref[...], f1.astype(bf))
        y1r = jnp.concatenate(
            [y1[o * 64:(o + 1) * 64] for o in range(9)], axis=1)
        z1 = _dot(y1r.astype(bf), m1_ref[...]) + b1_ref[...]

        f2 = jnp.concatenate(
            [z1 * jax.nn.sigmoid(z1)]
            + _bsplines(z1, _CONV_GRID, CONV_SPLINE_ORDER), axis=0)
        y2 = _dot(w2_ref[...], f2.astype(bf))
        y2r = jnp.concatenate(
            [y2[o * 128:(o + 1) * 128] for o in range(9)], axis=1)
        z2 = _dot(y2r.astype(bf), m2_ref[...]) + b2_ref[...]

        flats.append(jnp.concatenate(
            [z2[c * _BT:(c + 1) * _BT] for c in range(_C2)], axis=1))

    flat = jnp.concatenate(flats, axis=0)

    h1 = jnp.concatenate(
        [jnp.tanh(flat)] + _bsplines(flat, _KAN_GRID, KAN_SPLINE_ORDER), axis=1)
    h = _dot(h1.astype(bf), wk1_ref[...]) + bk1_ref[...]
    h2 = jnp.concatenate(
        [jnp.tanh(h)] + _bsplines(h, _KAN_GRID, KAN_SPLINE_ORDER), axis=1)
    o_ref[...] = _dot(h2.astype(bf), wk2_ref[...]) + bk2_ref[...]


def kernel(x, w1, m1, b1, w2, m2, b2, wk1, bk1, wk2, bk2):
    b, c_in, h, w = x.shape
    s0 = h * w
    x2 = x.reshape(b * c_in, s0)
    bf = jnp.float32

    weights = (
        w1.astype(bf),
        m1.reshape(9 * s0, m1.shape[-1]).astype(bf),
        b1,
        w2.astype(bf),
        m2.reshape(9 * m2.shape[1], m2.shape[-1]).astype(bf),
        b2,
        wk1.astype(bf), bk1,
        wk2.astype(bf), bk2,
    )

    def const_spec(a):
        return pl.BlockSpec(a.shape, lambda i, n=a.ndim: (0,) * n)

    step_rows = _NT * _BT * c_in
    out = pl.pallas_call(
        _kkan_body,
        grid=(b // (_NT * _BT),),
        out_shape=jax.ShapeDtypeStruct((b, 2), jnp.float32),
        in_specs=[pl.BlockSpec((step_rows, s0), lambda i: (i, 0))]
                 + [const_spec(a) for a in weights],
        out_specs=pl.BlockSpec((_NT * _BT, 2), lambda i: (i, 0)),
        compiler_params=pltpu.CompilerParams(
            dimension_semantics=("parallel",),
            vmem_limit_bytes=64 * 1024 * 1024),
    )(x2, *weights)
    return out


# speedup vs baseline: 2.3865x; 1.8482x over previous
import functools

import jax
import jax.numpy as jnp
from jax.experimental import pallas as pl
from jax.experimental.pallas import tpu as pltpu

CONV_GRID_SIZE, CONV_SPLINE_ORDER = 5, 3
KAN_GRID_SIZE, KAN_SPLINE_ORDER = 3, 3
_C0, _C1, _C2 = 2, 8, 16
_BT = 8
_NT = 8


def _ext_grid(num, k, lo=-1.0, hi=1.0):
    h = (hi - lo) / num
    return tuple(lo + (i - k) * h for i in range(num + 2 * k + 1))


_CONV_GRID = _ext_grid(CONV_GRID_SIZE, CONV_SPLINE_ORDER)
_KAN_GRID = _ext_grid(KAN_GRID_SIZE, KAN_SPLINE_ORDER)


def _bsplines(x, grid_pts, order):
    g = grid_pts
    step = [jnp.where(x >= gi, 1.0, 0.0) for gi in g]
    bases = [step[i] - step[i + 1] for i in range(len(g) - 1)]
    for k in range(1, order + 1):
        bases = [
            (x - g[i]) * (1.0 / (g[i + k] - g[i])) * bases[i]
            + (g[i + k + 1] - x) * (1.0 / (g[i + k + 1] - g[i + 1])) * bases[i + 1]
            for i in range(len(g) - 1 - k)
        ]
    return bases


def _dot(a, b):
    return jnp.dot(a, b, preferred_element_type=jnp.float32)


def _conv_feats(a):
    return jnp.concatenate(
        [a * jax.nn.sigmoid(a)] + _bsplines(a, _CONV_GRID, CONV_SPLINE_ORDER),
        axis=0)


def _kan_feats(a):
    return jnp.concatenate(
        [jnp.tanh(a)] + _bsplines(a, _KAN_GRID, KAN_SPLINE_ORDER), axis=1)


def _kkan_body(x_ref, w1_ref, m1_ref, b1_ref, w2_ref, m2_ref, b2_ref,
               wk1_ref, bk1_ref, wk2_ref, bk2_ref, o_ref):
    s1, s2 = 16 * 12, 8 * 6

    f1 = _conv_feats(x_ref[...])
    y1 = _dot(w1_ref[...], f1)
    z1 = b1_ref[...]
    for o in range(9):
        z1 = z1 + _dot(y1[o * 64:(o + 1) * 64],
                       m1_ref[o * _NT * s1:(o + 1) * _NT * s1])

    f2 = _conv_feats(z1)
    y2 = _dot(w2_ref[...], f2)
    z2 = b2_ref[...]
    for o in range(9):
        z2 = z2 + _dot(y2[o * 128:(o + 1) * 128],
                       m2_ref[o * _NT * s2:(o + 1) * _NT * s2])

    flat = jnp.concatenate(
        [jnp.concatenate(
            [z2[c * _BT:(c + 1) * _BT, t * _BT:(t + 1) * _BT]
             for c in range(_C2)], axis=1)
         for t in range(_NT)], axis=0)

    h = _dot(_kan_feats(flat), wk1_ref[...]) + bk1_ref[...]
    o_ref[...] = _dot(_kan_feats(h), wk2_ref[...]) + bk2_ref[...]


def kernel(x, w1, m1, b1, w2, m2, b2, wk1, bk1, wk2, bk2):
    b, c_in, h, w = x.shape
    s1 = h * w
    s2 = m2.shape[1]
    n1 = m1.shape[-1]
    n2 = m2.shape[-1]
    ntiles = b // _BT

    xt = x.reshape(ntiles, _BT * c_in, s1).transpose(1, 0, 2).reshape(
        _BT * c_in, ntiles * s1)

    eye = jnp.eye(_NT, dtype=jnp.float32)
    m1bd = jax.vmap(lambda m: jnp.kron(eye, m))(m1).reshape(9 * _NT * s1, _NT * n1)
    m2bd = jax.vmap(lambda m: jnp.kron(eye, m))(m2).reshape(9 * _NT * s2, _NT * n2)

    weights = (
        w1,
        m1bd,
        jnp.tile(b1, (1, _NT)),
        w2,
        m2bd,
        jnp.tile(b2, (1, _NT)),
        wk1, bk1,
        wk2, bk2,
    )

    def const_spec(a):
        return pl.BlockSpec(a.shape, lambda i, n=a.ndim: (0,) * n)

    out = pl.pallas_call(
        _kkan_body,
        grid=(ntiles // _NT,),
        out_shape=jax.ShapeDtypeStruct((b, 2), jnp.float32),
        in_specs=[pl.BlockSpec((_BT * c_in, _NT * s1), lambda i: (0, i))]
                 + [const_spec(a) for a in weights],
        out_specs=pl.BlockSpec((_NT * _BT, 2), lambda i: (i, 0)),
        compiler_params=pltpu.CompilerParams(
            dimension_semantics=("parallel",),
            vmem_limit_bytes=100 * 1024 * 1024),
    )(xt, *weights)
    return out


# speedup vs baseline: 2.5761x; 1.0795x over previous
import functools

import jax
import jax.numpy as jnp
from jax.experimental import pallas as pl
from jax.experimental.pallas import tpu as pltpu

CONV_GRID_SIZE, CONV_SPLINE_ORDER = 5, 3
KAN_GRID_SIZE, KAN_SPLINE_ORDER = 3, 3
_C0, _C1, _C2 = 2, 8, 16
_BT = 8
_NT = 8


def _ext_grid(num, k, lo=-1.0, hi=1.0):
    h = (hi - lo) / num
    return tuple(lo + (i - k) * h for i in range(num + 2 * k + 1))


_CONV_GRID = _ext_grid(CONV_GRID_SIZE, CONV_SPLINE_ORDER)
_KAN_GRID = _ext_grid(KAN_GRID_SIZE, KAN_SPLINE_ORDER)


def _bsplines(x, grid_pts, order):
    g = grid_pts
    d = [x - gi for gi in g]
    step = [jnp.where(di >= 0.0, 1.0, 0.0) for di in d]
    bases = [step[i] - step[i + 1] for i in range(len(g) - 1)]
    for k in range(1, order + 1):
        bases = [
            (1.0 / (g[i + k] - g[i])) * (d[i] * bases[i])
            - (1.0 / (g[i + k + 1] - g[i + 1])) * (d[i + k + 1] * bases[i + 1])
            for i in range(len(g) - 1 - k)
        ]
    return bases


def _dot(a, b):
    return jnp.dot(a, b, preferred_element_type=jnp.float32)


def _conv_feats(a):
    return jnp.concatenate(
        [a * jax.nn.sigmoid(a)] + _bsplines(a, _CONV_GRID, CONV_SPLINE_ORDER),
        axis=0)


def _kan_feats(a):
    return jnp.concatenate(
        [jnp.tanh(a)] + _bsplines(a, _KAN_GRID, KAN_SPLINE_ORDER), axis=1)


def _kkan_body(x_ref, w1_ref, m1_ref, b1_ref, w2_ref, m2_ref, b2_ref,
               wk1_ref, bk1_ref, wk2_ref, bk2_ref, o_ref, m1bd_ref):
    s1, s2 = 16 * 12, 8 * 6
    rt = _BT * _C0

    @pl.when(pl.program_id(1) == 0)
    def _build():
        m1bd_ref[...] = jnp.zeros_like(m1bd_ref)
        for o in range(9):
            blk = m1_ref[o * s1:(o + 1) * s1, :]
            for t in range(_NT):
                m1bd_ref[o * _NT * s1 + t * s1:
                         o * _NT * s1 + (t + 1) * s1,
                         t * 48:(t + 1) * 48] = blk

    x = x_ref[...]
    xt = jnp.concatenate(
        [x[t * rt:(t + 1) * rt, :] for t in range(_NT)], axis=1)

    f1 = _conv_feats(xt)
    y1 = _dot(w1_ref[...], f1)
    z1 = b1_ref[...]
    for o in range(9):
        z1 = z1 + _dot(y1[o * 64:(o + 1) * 64],
                       m1bd_ref[o * _NT * s1:(o + 1) * _NT * s1])

    f2 = _conv_feats(z1)
    y2 = _dot(w2_ref[...], f2)
    z2 = b2_ref[...]
    for o in range(9):
        z2 = z2 + _dot(y2[o * 128:(o + 1) * 128],
                       m2_ref[o * _NT * s2:(o + 1) * _NT * s2])

    flat = jnp.concatenate(
        [jnp.concatenate(
            [z2[c * _BT:(c + 1) * _BT, t * _BT:(t + 1) * _BT]
             for c in range(_C2)], axis=1)
         for t in range(_NT)], axis=0)

    h = _dot(_kan_feats(flat), wk1_ref[...]) + bk1_ref[...]
    o_ref[...] = _dot(_kan_feats(h), wk2_ref[...]) + bk2_ref[...]


def kernel(x, w1, m1, b1, w2, m2, b2, wk1, bk1, wk2, bk2):
    b, c_in, h, w = x.shape
    s1 = h * w
    s2 = m2.shape[1]
    n2 = m2.shape[-1]
    x2 = x.reshape(b * c_in, s1)
    nsteps = b // (_BT * _NT)
    ncore = 2
    nj = nsteps // ncore

    eye = jnp.eye(_NT, dtype=jnp.float32)
    m2bd = jax.vmap(lambda m: jnp.kron(eye, m))(m2).reshape(9 * _NT * s2, _NT * n2)

    weights = (
        w1,
        m1.reshape(9 * s1, m1.shape[-1]),
        jnp.tile(b1, (1, _NT)),
        w2,
        m2bd,
        jnp.tile(b2, (1, _NT)),
        wk1, bk1,
        wk2, bk2,
    )

    def const_spec(a):
        return pl.BlockSpec(a.shape, lambda c, j, n=a.ndim: (0,) * n)

    out = pl.pallas_call(
        _kkan_body,
        grid=(ncore, nj),
        out_shape=jax.ShapeDtypeStruct((b, 2), jnp.float32),
        in_specs=[pl.BlockSpec((_NT * _BT * c_in, s1),
                               lambda c, j: (c * nj + j, 0))]
                 + [const_spec(a) for a in weights],
        out_specs=pl.BlockSpec((_NT * _BT, 2), lambda c, j: (c * nj + j, 0)),
        scratch_shapes=[pltpu.VMEM((9 * _NT * s1, _NT * 48), jnp.float32)],
        compiler_params=pltpu.CompilerParams(
            dimension_semantics=("parallel", "arbitrary"),
            vmem_limit_bytes=100 * 1024 * 1024),
    )(x2, *weights)
    return out


# speedup vs baseline: 3.3750x; 1.3101x over previous
import functools

import jax
import jax.numpy as jnp
from jax.experimental import pallas as pl
from jax.experimental.pallas import tpu as pltpu

CONV_GRID_SIZE, CONV_SPLINE_ORDER = 5, 3
KAN_GRID_SIZE, KAN_SPLINE_ORDER = 3, 3
_C0, _C1, _C2 = 2, 8, 16
_BT = 8
_NT = 16
_NB = 8


def _ext_grid(num, k, lo=-1.0, hi=1.0):
    h = (hi - lo) / num
    return tuple(lo + (i - k) * h for i in range(num + 2 * k + 1))


_CONV_GRID = _ext_grid(CONV_GRID_SIZE, CONV_SPLINE_ORDER)
_KAN_GRID = _ext_grid(KAN_GRID_SIZE, KAN_SPLINE_ORDER)


def _bsplines(x, grid_pts, order):
    g = grid_pts
    d = [x - gi for gi in g]
    step = [jnp.where(di >= 0.0, 1.0, 0.0) for di in d]
    bases = [step[i] - step[i + 1] for i in range(len(g) - 1)]
    for k in range(1, order + 1):
        bases = [
            (1.0 / (g[i + k] - g[i])) * (d[i] * bases[i])
            - (1.0 / (g[i + k + 1] - g[i + 1])) * (d[i + k + 1] * bases[i + 1])
            for i in range(len(g) - 1 - k)
        ]
    return bases


def _dot(a, b):
    return jnp.dot(a, b, preferred_element_type=jnp.float32)


def _conv_feats(a):
    return jnp.concatenate(
        [a * jax.nn.sigmoid(a)] + _bsplines(a, _CONV_GRID, CONV_SPLINE_ORDER),
        axis=0)


def _kan_feats(a):
    return jnp.concatenate(
        [jnp.tanh(a)] + _bsplines(a, _KAN_GRID, KAN_SPLINE_ORDER), axis=1)


def _kkan_body(x_ref, w1_ref, m1_ref, b1_ref, w2_ref, m2_ref, b2_ref,
               wk1_ref, bk1_ref, wk2_ref, bk2_ref, o_ref, m1bd_ref):
    s1, s2 = 16 * 12, 8 * 6
    rt = _BT * _C0

    @pl.when(pl.program_id(1) == 0)
    def _build():
        m1bd_ref[...] = jnp.zeros_like(m1bd_ref)
        for o in range(9):
            blk = m1_ref[o * s1:(o + 1) * s1, :]
            for t in range(_NB):
                m1bd_ref[o * _NB * s1 + t * s1:
                         o * _NB * s1 + (t + 1) * s1,
                         t * 48:(t + 1) * 48] = blk

    x = x_ref[...]
    xt = jnp.concatenate(
        [x[t * rt:(t + 1) * rt, :] for t in range(_NT)], axis=1)

    f1 = _conv_feats(xt)
    y1 = _dot(w1_ref[...], f1)
    h1 = _NB * s1
    acc1 = None
    for o in range(9):
        yo = y1[o * 64:(o + 1) * 64]
        ys = jnp.concatenate([yo[:, :h1], yo[:, h1:]], axis=0)
        d = _dot(ys, m1bd_ref[o * h1:(o + 1) * h1])
        acc1 = d if acc1 is None else acc1 + d
    z1 = jnp.concatenate([acc1[:64], acc1[64:]], axis=1) + b1_ref[...]

    f2 = _conv_feats(z1)
    y2 = _dot(w2_ref[...], f2)
    h2 = _NB * s2
    acc2 = None
    for o in range(9):
        yo = y2[o * 128:(o + 1) * 128]
        ys = jnp.concatenate([yo[:, :h2], yo[:, h2:]], axis=0)
        d = _dot(ys, m2_ref[o * h2:(o + 1) * h2])
        acc2 = d if acc2 is None else acc2 + d
    z2 = jnp.concatenate([acc2[:128], acc2[128:]], axis=1) + b2_ref[...]

    flat = jnp.concatenate(
        [jnp.concatenate(
            [z2[c * _BT:(c + 1) * _BT, t * _BT:(t + 1) * _BT]
             for c in range(_C2)], axis=1)
         for t in range(_NT)], axis=0)

    h = _dot(_kan_feats(flat), wk1_ref[...]) + bk1_ref[...]
    o_ref[...] = _dot(_kan_feats(h), wk2_ref[...]) + bk2_ref[...]


def kernel(x, w1, m1, b1, w2, m2, b2, wk1, bk1, wk2, bk2):
    b, c_in, h, w = x.shape
    s1 = h * w
    s2 = m2.shape[1]
    n2 = m2.shape[-1]
    x2 = x.reshape(b * c_in, s1)
    nsteps = b // (_BT * _NT)
    ncore = 2
    nj = nsteps // ncore

    eye = jnp.eye(_NB, dtype=jnp.float32)
    m2bd = jax.vmap(lambda m: jnp.kron(eye, m))(m2).reshape(9 * _NB * s2, _NB * n2)

    weights = (
        w1,
        m1.reshape(9 * s1, m1.shape[-1]),
        jnp.tile(b1, (1, _NT)),
        w2,
        m2bd,
        jnp.tile(b2, (1, _NT)),
        wk1, bk1,
        wk2, bk2,
    )

    def const_spec(a):
        return pl.BlockSpec(a.shape, lambda c, j, n=a.ndim: (0,) * n)

    out = pl.pallas_call(
        _kkan_body,
        grid=(ncore, nj),
        out_shape=jax.ShapeDtypeStruct((b, 2), jnp.float32),
        in_specs=[pl.BlockSpec((_NT * _BT * c_in, s1),
                               lambda c, j: (c * nj + j, 0))]
                 + [const_spec(a) for a in weights],
        out_specs=pl.BlockSpec((_NT * _BT, 2), lambda c, j: (c * nj + j, 0)),
        scratch_shapes=[pltpu.VMEM((9 * _NB * s1, _NB * 48), jnp.float32)],
        compiler_params=pltpu.CompilerParams(
            dimension_semantics=("parallel", "arbitrary"),
            vmem_limit_bytes=100 * 1024 * 1024),
    )(x2, *weights)
    return out


# speedup vs baseline: 3.8963x; 1.1545x over previous
import functools

import jax
import jax.numpy as jnp
from jax.experimental import pallas as pl
from jax.experimental.pallas import tpu as pltpu

CONV_GRID_SIZE, CONV_SPLINE_ORDER = 5, 3
KAN_GRID_SIZE, KAN_SPLINE_ORDER = 3, 3
_C0, _C1, _C2 = 2, 8, 16
_BT = 8
_NT = 32
_NB = 8


def _ext_grid(num, k, lo=-1.0, hi=1.0):
    h = (hi - lo) / num
    return tuple(lo + (i - k) * h for i in range(num + 2 * k + 1))


_CONV_GRID = _ext_grid(CONV_GRID_SIZE, CONV_SPLINE_ORDER)
_KAN_GRID = _ext_grid(KAN_GRID_SIZE, KAN_SPLINE_ORDER)


def _bsplines(x, grid_pts, order):
    g = grid_pts
    d = [x - gi for gi in g]
    step = [jnp.where(di >= 0.0, 1.0, 0.0) for di in d]
    bases = [step[i] - step[i + 1] for i in range(len(g) - 1)]
    for k in range(1, order + 1):
        bases = [
            (1.0 / (g[i + k] - g[i])) * (d[i] * bases[i])
            - (1.0 / (g[i + k + 1] - g[i + 1])) * (d[i + k + 1] * bases[i + 1])
            for i in range(len(g) - 1 - k)
        ]
    return bases


def _dot(a, b):
    return jnp.dot(a, b, preferred_element_type=jnp.float32)


def _conv_feats(a):
    return jnp.concatenate(
        [a * jax.nn.sigmoid(a)] + _bsplines(a, _CONV_GRID, CONV_SPLINE_ORDER),
        axis=0)


def _kan_feats(a):
    return jnp.concatenate(
        [jnp.tanh(a)] + _bsplines(a, _KAN_GRID, KAN_SPLINE_ORDER), axis=1)


def _kkan_body(x_ref, w1_ref, m1_ref, b1_ref, w2_ref, m2_ref, b2_ref,
               wk1_ref, bk1_ref, wk2_ref, bk2_ref, o_ref, m1bd_ref):
    s1, s2 = 16 * 12, 8 * 6
    rt = _BT * _C0

    @pl.when(pl.program_id(1) == 0)
    def _build():
        m1bd_ref[...] = jnp.zeros_like(m1bd_ref)
        for o in range(9):
            blk = m1_ref[o * s1:(o + 1) * s1, :]
            for t in range(_NB):
                m1bd_ref[o * _NB * s1 + t * s1:
                         o * _NB * s1 + (t + 1) * s1,
                         t * 48:(t + 1) * 48] = blk

    x = x_ref[...]
    xt = jnp.concatenate(
        [x[t * rt:(t + 1) * rt, :] for t in range(_NT)], axis=1)

    f1 = _conv_feats(xt)
    y1 = _dot(w1_ref[...], f1)
    nq = _NT // _NB
    h1 = _NB * s1
    acc1 = None
    for o in range(9):
        yo = y1[o * 64:(o + 1) * 64]
        ys = jnp.concatenate(
            [yo[:, q * h1:(q + 1) * h1] for q in range(nq)], axis=0)
        d = _dot(ys, m1bd_ref[o * h1:(o + 1) * h1])
        acc1 = d if acc1 is None else acc1 + d
    z1 = jnp.concatenate(
        [acc1[q * 64:(q + 1) * 64] for q in range(nq)], axis=1) + b1_ref[...]

    f2 = _conv_feats(z1)
    y2 = _dot(w2_ref[...], f2)
    h2 = _NB * s2
    acc2 = None
    for o in range(9):
        yo = y2[o * 128:(o + 1) * 128]
        ys = jnp.concatenate(
            [yo[:, q * h2:(q + 1) * h2] for q in range(nq)], axis=0)
        d = _dot(ys, m2_ref[o * h2:(o + 1) * h2])
        acc2 = d if acc2 is None else acc2 + d
    z2 = jnp.concatenate(
        [acc2[q * 128:(q + 1) * 128] for q in range(nq)], axis=1) + b2_ref[...]

    flat = jnp.concatenate(
        [jnp.concatenate(
            [z2[c * _BT:(c + 1) * _BT, t * _BT:(t + 1) * _BT]
             for c in range(_C2)], axis=1)
         for t in range(_NT)], axis=0)

    h = _dot(_kan_feats(flat), wk1_ref[...]) + bk1_ref[...]
    o_ref[...] = _dot(_kan_feats(h), wk2_ref[...]) + bk2_ref[...]


def kernel(x, w1, m1, b1, w2, m2, b2, wk1, bk1, wk2, bk2):
    b, c_in, h, w = x.shape
    s1 = h * w
    s2 = m2.shape[1]
    n2 = m2.shape[-1]
    x2 = x.reshape(b * c_in, s1)
    nsteps = b // (_BT * _NT)
    ncore = 2
    nj = nsteps // ncore

    eye = jnp.eye(_NB, dtype=jnp.float32)
    m2bd = jax.vmap(lambda m: jnp.kron(eye, m))(m2).reshape(9 * _NB * s2, _NB * n2)

    weights = (
        w1,
        m1.reshape(9 * s1, m1.shape[-1]),
        jnp.tile(b1, (1, _NT)),
        w2,
        m2bd,
        jnp.tile(b2, (1, _NT)),
        wk1, bk1,
        wk2, bk2,
    )

    def const_spec(a):
        return pl.BlockSpec(a.shape, lambda c, j, n=a.ndim: (0,) * n)

    out = pl.pallas_call(
        _kkan_body,
        grid=(ncore, nj),
        out_shape=jax.ShapeDtypeStruct((b, 2), jnp.float32),
        in_specs=[pl.BlockSpec((_NT * _BT * c_in, s1),
                               lambda c, j: (c * nj + j, 0))]
                 + [const_spec(a) for a in weights],
        out_specs=pl.BlockSpec((_NT * _BT, 2), lambda c, j: (c * nj + j, 0)),
        scratch_shapes=[pltpu.VMEM((9 * _NB * s1, _NB * 48), jnp.float32)],
        compiler_params=pltpu.CompilerParams(
            dimension_semantics=("parallel", "arbitrary"),
            vmem_limit_bytes=100 * 1024 * 1024),
    )(x2, *weights)
    return out


# speedup vs baseline: 4.0482x; 1.0390x over previous
import functools

import jax
import jax.numpy as jnp
from jax.experimental import pallas as pl
from jax.experimental.pallas import tpu as pltpu

CONV_GRID_SIZE, CONV_SPLINE_ORDER = 5, 3
KAN_GRID_SIZE, KAN_SPLINE_ORDER = 3, 3
_C0, _C1, _C2 = 2, 8, 16
_BT = 8
_NT = 64
_NB = 8


def _ext_grid(num, k, lo=-1.0, hi=1.0):
    h = (hi - lo) / num
    return tuple(lo + (i - k) * h for i in range(num + 2 * k + 1))


_CONV_GRID = _ext_grid(CONV_GRID_SIZE, CONV_SPLINE_ORDER)
_KAN_GRID = _ext_grid(KAN_GRID_SIZE, KAN_SPLINE_ORDER)


def _bsplines(x, grid_pts, order):
    g = grid_pts
    d = [x - gi for gi in g]
    step = [jnp.where(di >= 0.0, 1.0, 0.0) for di in d]
    bases = [step[i] - step[i + 1] for i in range(len(g) - 1)]
    for k in range(1, order + 1):
        bases = [
            (1.0 / (g[i + k] - g[i])) * (d[i] * bases[i])
            - (1.0 / (g[i + k + 1] - g[i + 1])) * (d[i + k + 1] * bases[i + 1])
            for i in range(len(g) - 1 - k)
        ]
    return bases


def _dot(a, b):
    return jnp.dot(a, b, preferred_element_type=jnp.float32)


def _conv_feats(a):
    return jnp.concatenate(
        [a * jax.nn.sigmoid(a)] + _bsplines(a, _CONV_GRID, CONV_SPLINE_ORDER),
        axis=0)


def _kan_feats(a):
    return jnp.concatenate(
        [jnp.tanh(a)] + _bsplines(a, _KAN_GRID, KAN_SPLINE_ORDER), axis=1)


def _kkan_body(x_ref, w1_ref, m1_ref, b1_ref, w2_ref, m2_ref, b2_ref,
               wk1_ref, bk1_ref, wk2_ref, bk2_ref, o_ref, m1bd_ref):
    s1, s2 = 16 * 12, 8 * 6
    rt = _BT * _C0

    @pl.when(pl.program_id(1) == 0)
    def _build():
        m1bd_ref[...] = jnp.zeros_like(m1bd_ref)
        for o in range(9):
            blk = m1_ref[o * s1:(o + 1) * s1, :]
            for t in range(_NB):
                m1bd_ref[o * _NB * s1 + t * s1:
                         o * _NB * s1 + (t + 1) * s1,
                         t * 48:(t + 1) * 48] = blk

    x = x_ref[...]
    xt = jnp.concatenate(
        [x[t * rt:(t + 1) * rt, :] for t in range(_NT)], axis=1)

    f1 = _conv_feats(xt)
    y1 = _dot(w1_ref[...], f1)
    nq = _NT // _NB
    h1 = _NB * s1
    acc1 = None
    for o in range(9):
        yo = y1[o * 64:(o + 1) * 64]
        ys = jnp.concatenate(
            [yo[:, q * h1:(q + 1) * h1] for q in range(nq)], axis=0)
        d = _dot(ys, m1bd_ref[o * h1:(o + 1) * h1])
        acc1 = d if acc1 is None else acc1 + d
    z1 = jnp.concatenate(
        [acc1[q * 64:(q + 1) * 64] for q in range(nq)], axis=1) + b1_ref[...]

    f2 = _conv_feats(z1)
    y2 = _dot(w2_ref[...], f2)
    h2 = _NB * s2
    acc2 = None
    for o in range(9):
        yo = y2[o * 128:(o + 1) * 128]
        ys = jnp.concatenate(
            [yo[:, q * h2:(q + 1) * h2] for q in range(nq)], axis=0)
        d = _dot(ys, m2_ref[o * h2:(o + 1) * h2])
        acc2 = d if acc2 is None else acc2 + d
    z2 = jnp.concatenate(
        [acc2[q * 128:(q + 1) * 128] for q in range(nq)], axis=1) + b2_ref[...]

    flat = jnp.concatenate(
        [jnp.concatenate(
            [z2[c * _BT:(c + 1) * _BT, t * _BT:(t + 1) * _BT]
             for c in range(_C2)], axis=1)
         for t in range(_NT)], axis=0)

    h = _dot(_kan_feats(flat), wk1_ref[...]) + bk1_ref[...]
    o_ref[...] = _dot(_kan_feats(h), wk2_ref[...]) + bk2_ref[...]


def kernel(x, w1, m1, b1, w2, m2, b2, wk1, bk1, wk2, bk2):
    b, c_in, h, w = x.shape
    s1 = h * w
    s2 = m2.shape[1]
    n2 = m2.shape[-1]
    x2 = x.reshape(b * c_in, s1)
    nsteps = b // (_BT * _NT)
    ncore = 2
    nj = nsteps // ncore

    eye = jnp.eye(_NB, dtype=jnp.float32)
    m2bd = jax.vmap(lambda m: jnp.kron(eye, m))(m2).reshape(9 * _NB * s2, _NB * n2)

    weights = (
        w1,
        m1.reshape(9 * s1, m1.shape[-1]),
        jnp.tile(b1, (1, _NT)),
        w2,
        m2bd,
        jnp.tile(b2, (1, _NT)),
        wk1, bk1,
        wk2, bk2,
    )

    def const_spec(a):
        return pl.BlockSpec(a.shape, lambda c, j, n=a.ndim: (0,) * n)

    out = pl.pallas_call(
        _kkan_body,
        grid=(ncore, nj),
        out_shape=jax.ShapeDtypeStruct((b, 2), jnp.float32),
        in_specs=[pl.BlockSpec((_NT * _BT * c_in, s1),
                               lambda c, j: (c * nj + j, 0))]
                 + [const_spec(a) for a in weights],
        out_specs=pl.BlockSpec((_NT * _BT, 2), lambda c, j: (c * nj + j, 0)),
        scratch_shapes=[pltpu.VMEM((9 * _NB * s1, _NB * 48), jnp.float32)],
        compiler_params=pltpu.CompilerParams(
            dimension_semantics=("parallel", "arbitrary"),
            vmem_limit_bytes=100 * 1024 * 1024),
    )(x2, *weights)
    return out
